# baseline (speedup 1.0000x reference)
import functools

import jax
import jax.numpy as jnp
from jax import lax
from jax.experimental import pallas as pl
from jax.experimental.pallas import tpu as pltpu
from jax.experimental.pallas import tpu_sc as plsc

NC = 2
NS = 16
L = 16
NW = NC * NS
CH = 128


def _sc_degree(n_pad, jd):
    mesh = plsc.VectorSubcoreMesh(core_axis_name="c", subcore_axis_name="s")
    gstep = 8

    @functools.partial(
        pl.kernel,
        out_type=jax.ShapeDtypeStruct((NC, NS, n_pad), jnp.float32),
        mesh=mesh,
        scratch_types=[
            pltpu.VMEM((gstep, CH), jnp.int32),
            pltpu.VMEM((n_pad,), jnp.float32),
        ],
        compiler_params=pltpu.CompilerParams(needs_layout_passes=False),
    )
    def deg_kernel(pk_hbm, out_hbm, stage_v, deg_v):
        c = lax.axis_index("c")
        s = lax.axis_index("s")
        w = c * NS + s
        zeros16 = jnp.zeros((L,), jnp.float32)

        def zb(i, carry):
            deg_v[pl.ds(i * L, L)] = zeros16
            return carry

        lax.fori_loop(0, n_pad // L, zb, 0)
        ones16 = jnp.ones((L,), jnp.float32)

        def body(r, carry):
            for k in range(CH // L):
                p = stage_v[r, pl.ds(k * L, L)]
                plsc.addupdate_scatter(
                    deg_v, [lax.shift_right_logical(p, 16)], ones16)
            return carry

        for g in range(0, jd, gstep):
            pltpu.sync_copy(pk_hbm.at[w, pl.ds(g, gstep)], stage_v)
            lax.fori_loop(0, gstep, body, 0)
        pltpu.sync_copy(deg_v, out_hbm.at[c, s])

    return deg_kernel


def _sc_scatter(n_pad, jw, d):
    mesh = plsc.VectorSubcoreMesh(core_axis_name="c", subcore_axis_name="s")
    rt = n_pad // NS

    @functools.partial(
        pl.kernel,
        out_type=jax.ShapeDtypeStruct((NC, n_pad, d), jnp.float32),
        mesh=mesh,
        scratch_types=[
            pltpu.VMEM((2, CH), jnp.int32),
            pltpu.VMEM((2, CH), jnp.int32),
            pltpu.VMEM((2, CH), jnp.int32),
            pltpu.VMEM((2, CH, d), jnp.float32),
            pltpu.VMEM_SHARED((n_pad, d), jnp.float32),
            pltpu.SemaphoreType.DMA,
            pltpu.SemaphoreType.DMA,
            pltpu.SemaphoreType.DMA,
        ],
    )
    def scat_kernel(hp_hbm, pk_hbm, out_hbm,
                    pk_v, su_v, du_v, rows_v, acc_sh, gsem, ssem, psem):
        c = lax.axis_index("c")
        s = lax.axis_index("s")

        zeros16 = jnp.zeros((L,), jnp.float32)

        def zb(i, carry):
            for k in range(d // L):
                rows_v[1, i, pl.ds(k * L, L)] = zeros16
            return carry

        lax.fori_loop(0, CH, zb, 0)

        def zb2(i, carry):
            pltpu.sync_copy(
                rows_v.at[1],
                acc_sh.at[pl.ds(pl.multiple_of(s * rt + i * CH, 8), CH)])
            return carry

        lax.fori_loop(0, rt // CH, zb2, 0)
        if rt % CH:
            pltpu.sync_copy(
                rows_v.at[1, pl.ds(0, rt % CH)],
                acc_sh.at[pl.ds(
                    pl.multiple_of(s * rt + (rt // CH) * CH, 8), rt % CH)])
        plsc.subcore_barrier()

        mask16 = jnp.full((L,), 0xFFFF, jnp.int32)

        def stage_pk(j, b):
            pltpu.async_copy(pk_hbm.at[c, s, j], pk_v.at[b], psem)

        def wait_pk(j, b):
            pltpu.make_async_copy(pk_hbm.at[c, s, j], pk_v.at[b], psem).wait()

        def unpack(j, b):
            def ub(i, carry):
                p = pk_v[b, pl.ds(i * L, L)]
                su_v[b, pl.ds(i * L, L)] = jnp.bitwise_and(p, mask16)
                du_v[b, pl.ds(i * L, L)] = lax.shift_right_logical(p, 16)
                return carry

            lax.fori_loop(0, CH // L, ub, 0)

        stage_pk(0, 0)
        wait_pk(0, 0)
        unpack(0, 0)
        pltpu.async_copy(hp_hbm.at[su_v.at[0]], rows_v.at[0], gsem)
        if jw > 1:
            stage_pk(1, 1)

        def body(j, carry):
            b = lax.rem(j, 2)
            pltpu.make_async_copy(hp_hbm.at[su_v.at[b]],
                                  rows_v.at[b], gsem).wait()
            pltpu.async_copy(rows_v.at[b], acc_sh.at[du_v.at[b]], ssem,
                             add=True)

            @pl.when(j >= 1)
            def _():
                pltpu.make_async_copy(rows_v.at[1 - b],
                                      acc_sh.at[du_v.at[1 - b]], ssem).wait()

            @pl.when(j + 1 < jw)
            def _():
                wait_pk(j + 1, 1 - b)
                unpack(j + 1, 1 - b)
                pltpu.async_copy(hp_hbm.at[su_v.at[1 - b]],
                                 rows_v.at[1 - b], gsem)

            @pl.when(j + 2 < jw)
            def _():
                stage_pk(j + 2, b)

            return carry

        lax.fori_loop(0, jw, body, 0)
        pltpu.make_async_copy(rows_v.at[(jw - 1) % 2],
                              acc_sh.at[du_v.at[(jw - 1) % 2]], ssem).wait()
        plsc.subcore_barrier()
        pltpu.sync_copy(
            acc_sh.at[pl.ds(pl.multiple_of(s * rt, 8), rt)],
            out_hbm.at[c, pl.ds(pl.multiple_of(s * rt, 8), rt)])

    return scat_kernel


def _tc_dinv(deg_p, n_pad):
    def body(dp_ref, o_ref):
        deg = jnp.sum(dp_ref[...], axis=(0, 1)) + 1.0
        o_ref[...] = lax.rsqrt(deg).reshape(1, n_pad)

    return pl.pallas_call(
        body,
        grid=(1,),
        in_specs=[pl.BlockSpec((NC, NS, n_pad), lambda i: (0, 0, 0))],
        out_specs=pl.BlockSpec((1, n_pad), lambda i: (0, 0)),
        out_shape=jax.ShapeDtypeStruct((1, n_pad), jnp.float32),
    )(deg_p)


def _tc_first(dinv, x, w, blk):
    n, d = x.shape

    def body(dv_ref, x_ref, w_ref, o_ref):
        h = jnp.dot(x_ref[...], w_ref[...], preferred_element_type=jnp.float32)
        o_ref[...] = h * dv_ref[...]

    return pl.pallas_call(
        body,
        grid=(n // blk,),
        in_specs=[
            pl.BlockSpec((blk, 1), lambda i: (i, 0)),
            pl.BlockSpec((blk, d), lambda i: (i, 0)),
            pl.BlockSpec((d, d), lambda i: (0, 0)),
        ],
        out_specs=pl.BlockSpec((blk, d), lambda i: (i, 0)),
        out_shape=jax.ShapeDtypeStruct((n, d), jnp.float32),
    )(dinv, x, w)


def _tc_mid(dinv, p, hp, b, a, w, blk):
    n, d = hp.shape

    def body(dv_ref, p_ref, hp_ref, b_ref, a_ref, w_ref, o_ref):
        dc = dv_ref[...]
        t = dc * (p_ref[0] + p_ref[1] + hp_ref[...]) + b_ref[...]
        u = jnp.where(t >= 0, t, a_ref[...] * t)
        h = jnp.dot(u, w_ref[...], preferred_element_type=jnp.float32)
        o_ref[...] = h * dc

    return pl.pallas_call(
        body,
        grid=(n // blk,),
        in_specs=[
            pl.BlockSpec((blk, 1), lambda i: (i, 0)),
            pl.BlockSpec((NC, blk, d), lambda i: (0, i, 0)),
            pl.BlockSpec((blk, d), lambda i: (i, 0)),
            pl.BlockSpec((1, d), lambda i: (0, 0)),
            pl.BlockSpec((1, 1), lambda i: (0, 0)),
            pl.BlockSpec((d, d), lambda i: (0, 0)),
        ],
        out_specs=pl.BlockSpec((blk, d), lambda i: (i, 0)),
        out_shape=jax.ShapeDtypeStruct((n, d), jnp.float32),
    )(dinv, p, hp, b, a, w)


def _tc_last(dinv, p, hp, b, a, blk):
    n, d = hp.shape

    def body(dv_ref, p_ref, hp_ref, b_ref, a_ref, o_ref):
        t = dv_ref[...] * (p_ref[0] + p_ref[1] + hp_ref[...]) + b_ref[...]
        o_ref[...] = jnp.where(t >= 0, t, a_ref[...] * t)

    return pl.pallas_call(
        body,
        grid=(n // blk,),
        in_specs=[
            pl.BlockSpec((blk, 1), lambda i: (i, 0)),
            pl.BlockSpec((NC, blk, d), lambda i: (0, i, 0)),
            pl.BlockSpec((blk, d), lambda i: (i, 0)),
            pl.BlockSpec((1, d), lambda i: (0, 0)),
            pl.BlockSpec((1, 1), lambda i: (0, 0)),
        ],
        out_specs=pl.BlockSpec((blk, d), lambda i: (i, 0)),
        out_shape=jax.ShapeDtypeStruct((n, d), jnp.float32),
    )(dinv, p, hp, b, a)


def kernel(x, edge_index, W1, b1, a1, W2, b2, a2):
    n, d = x.shape
    e = edge_index.shape[1]
    src = edge_index[0].astype(jnp.int32)
    dst = edge_index[1].astype(jnp.int32)

    nch = -(-e // (CH * NW * 8)) * NW * 8
    ep = nch * CH
    jw = nch // NW
    n_pad = (n // 128 + 2) * 128

    pad_dst = n + lax.rem(jnp.arange(ep - e, dtype=jnp.int32), n_pad - n)
    pk = jnp.concatenate([src | (dst << 16), pad_dst << 16])
    pk_w = pk.reshape(NW, jw, CH)
    pk_cs = pk.reshape(NC, NS, jw, CH)

    blk = 1000 if n % 1000 == 0 else 8
    assert n % blk == 0

    deg_p = _sc_degree(n_pad, jw)(pk_w)
    dinv = jnp.swapaxes(_tc_dinv(deg_p, n_pad), 0, 1)[:n]

    scat = _sc_scatter(n_pad, jw, d)
    hp1 = _tc_first(dinv, x, W1, blk)
    p1 = scat(hp1, pk_cs)
    hp2 = _tc_mid(dinv, p1, hp1, b1.reshape(1, d), a1.reshape(1, 1), W2, blk)
    p2 = scat(hp2, pk_cs)
    return _tc_last(dinv, p2, hp2, b2.reshape(1, d), a2.reshape(1, 1), blk)


# speedup vs baseline: 3.0481x; 3.0481x over previous
import functools

import jax
import jax.numpy as jnp
from jax import lax
from jax.experimental import pallas as pl
from jax.experimental.pallas import tpu as pltpu
from jax.experimental.pallas import tpu_sc as plsc

NC = 2
NS = 16
L = 16
NW = NC * NS
CH = 128


def _sc_degree(n_pad, jd):
    mesh = plsc.VectorSubcoreMesh(core_axis_name="c", subcore_axis_name="s")
    gstep = 8

    @functools.partial(
        pl.kernel,
        out_type=jax.ShapeDtypeStruct((NC, NS, n_pad), jnp.float32),
        mesh=mesh,
        scratch_types=[
            pltpu.VMEM((gstep, CH), jnp.int32),
            pltpu.VMEM((n_pad,), jnp.float32),
        ],
        compiler_params=pltpu.CompilerParams(needs_layout_passes=False),
    )
    def deg_kernel(pk_hbm, out_hbm, stage_v, deg_v):
        c = lax.axis_index("c")
        s = lax.axis_index("s")
        w = c * NS + s
        zeros16 = jnp.zeros((L,), jnp.float32)

        def zb(i, carry):
            deg_v[pl.ds(i * L, L)] = zeros16
            return carry

        lax.fori_loop(0, n_pad // L, zb, 0)
        ones16 = jnp.ones((L,), jnp.float32)

        def body(r, carry):
            for k in range(CH // L):
                p = stage_v[r, pl.ds(k * L, L)]
                plsc.addupdate_scatter(
                    deg_v, [lax.shift_right_logical(p, 16)], ones16)
            return carry

        for g in range(0, jd, gstep):
            pltpu.sync_copy(pk_hbm.at[w, pl.ds(g, gstep)], stage_v)
            lax.fori_loop(0, gstep, body, 0)
        pltpu.sync_copy(deg_v, out_hbm.at[c, s])

    return deg_kernel


def _sc_scatter(n_pad, jw, d):
    mesh = plsc.VectorSubcoreMesh(core_axis_name="c", subcore_axis_name="s")
    rt = n_pad // NS

    @functools.partial(
        pl.kernel,
        out_type=jax.ShapeDtypeStruct((NC, n_pad, d), jnp.float32),
        mesh=mesh,
        scratch_types=[
            pltpu.VMEM((2, CH), jnp.int32),
            pltpu.VMEM((2, CH), jnp.int32),
            pltpu.VMEM((2, CH), jnp.int32),
            pltpu.VMEM((2, CH, d), jnp.float32),
            pltpu.VMEM_SHARED((n_pad, d), jnp.float32),
            pltpu.SemaphoreType.DMA,
            pltpu.SemaphoreType.DMA,
            pltpu.SemaphoreType.DMA,
        ],
    )
    def scat_kernel(hp_hbm, pk_hbm, out_hbm,
                    pk_v, su_v, du_v, rows_v, acc_sh, gsem, ssem, psem):
        c = lax.axis_index("c")
        s = lax.axis_index("s")

        zeros16 = jnp.zeros((L,), jnp.float32)

        def zb(i, carry):
            for k in range(d // L):
                rows_v[1, i, pl.ds(k * L, L)] = zeros16
            return carry

        lax.fori_loop(0, CH, zb, 0)

        def zb2(i, carry):
            pltpu.sync_copy(
                rows_v.at[1],
                acc_sh.at[pl.ds(pl.multiple_of(s * rt + i * CH, 8), CH)])
            return carry

        lax.fori_loop(0, rt // CH, zb2, 0)
        if rt % CH:
            pltpu.sync_copy(
                rows_v.at[1, pl.ds(0, rt % CH)],
                acc_sh.at[pl.ds(
                    pl.multiple_of(s * rt + (rt // CH) * CH, 8), rt % CH)])
        plsc.subcore_barrier()

        mask16 = jnp.full((L,), 0xFFFF, jnp.int32)

        def stage_pk(j, b):
            pltpu.async_copy(pk_hbm.at[c, s, j], pk_v.at[b], psem)

        def wait_pk(j, b):
            pltpu.make_async_copy(pk_hbm.at[c, s, j], pk_v.at[b], psem).wait()

        def unpack(j, b):
            def ub(i, carry):
                p = pk_v[b, pl.ds(i * L, L)]
                su_v[b, pl.ds(i * L, L)] = jnp.bitwise_and(p, mask16)
                du_v[b, pl.ds(i * L, L)] = lax.shift_right_logical(p, 16)
                return carry

            lax.fori_loop(0, CH // L, ub, 0)

        stage_pk(0, 0)
        wait_pk(0, 0)
        unpack(0, 0)
        pltpu.async_copy(hp_hbm.at[su_v.at[0]], rows_v.at[0], gsem)
        if jw > 1:
            stage_pk(1, 1)

        def body(j, carry):
            b = lax.rem(j, 2)
            pltpu.make_async_copy(hp_hbm.at[su_v.at[b]],
                                  rows_v.at[b], gsem).wait()
            pltpu.async_copy(rows_v.at[b], acc_sh.at[du_v.at[b]], ssem,
                             add=True)

            @pl.when(j >= 1)
            def _():
                pltpu.make_async_copy(rows_v.at[1 - b],
                                      acc_sh.at[du_v.at[1 - b]], ssem).wait()

            @pl.when(j + 1 < jw)
            def _():
                wait_pk(j + 1, 1 - b)
                unpack(j + 1, 1 - b)
                pltpu.async_copy(hp_hbm.at[su_v.at[1 - b]],
                                 rows_v.at[1 - b], gsem)

            @pl.when(j + 2 < jw)
            def _():
                stage_pk(j + 2, b)

            return carry

        lax.fori_loop(0, jw, body, 0)
        pltpu.make_async_copy(rows_v.at[(jw - 1) % 2],
                              acc_sh.at[du_v.at[(jw - 1) % 2]], ssem).wait()
        plsc.subcore_barrier()
        pltpu.sync_copy(
            acc_sh.at[pl.ds(pl.multiple_of(s * rt, 8), rt)],
            out_hbm.at[c, pl.ds(pl.multiple_of(s * rt, 8), rt)])

    return scat_kernel


def _tc_dinv(deg_p, n_pad):
    def body(dp_ref, o_ref):
        deg = jnp.sum(dp_ref[...], axis=(0, 1)) + 1.0
        o_ref[...] = lax.rsqrt(deg).reshape(1, n_pad)

    return pl.pallas_call(
        body,
        grid=(1,),
        in_specs=[pl.BlockSpec((NC, NS, n_pad), lambda i: (0, 0, 0))],
        out_specs=pl.BlockSpec((1, n_pad), lambda i: (0, 0)),
        out_shape=jax.ShapeDtypeStruct((1, n_pad), jnp.float32),
    )(deg_p)


def _tc_first(dinv, x, w, blk):
    n, d = x.shape

    def body(dv_ref, x_ref, w_ref, o_ref):
        h = jnp.dot(x_ref[...], w_ref[...], preferred_element_type=jnp.float32)
        o_ref[...] = h * dv_ref[...]

    return pl.pallas_call(
        body,
        grid=(n // blk,),
        in_specs=[
            pl.BlockSpec((blk, 1), lambda i: (i, 0)),
            pl.BlockSpec((blk, d), lambda i: (i, 0)),
            pl.BlockSpec((d, d), lambda i: (0, 0)),
        ],
        out_specs=pl.BlockSpec((blk, d), lambda i: (i, 0)),
        out_shape=jax.ShapeDtypeStruct((n, d), jnp.float32),
    )(dinv, x, w)


def _tc_mid(dinv, p, hp, b, a, w, blk):
    n, d = hp.shape

    def body(dv_ref, p_ref, hp_ref, b_ref, a_ref, w_ref, o_ref):
        dc = dv_ref[...]
        t = dc * (p_ref[0] + p_ref[1] + hp_ref[...]) + b_ref[...]
        u = jnp.where(t >= 0, t, a_ref[...] * t)
        h = jnp.dot(u, w_ref[...], preferred_element_type=jnp.float32)
        o_ref[...] = h * dc

    return pl.pallas_call(
        body,
        grid=(n // blk,),
        in_specs=[
            pl.BlockSpec((blk, 1), lambda i: (i, 0)),
            pl.BlockSpec((NC, blk, d), lambda i: (0, i, 0)),
            pl.BlockSpec((blk, d), lambda i: (i, 0)),
            pl.BlockSpec((1, d), lambda i: (0, 0)),
            pl.BlockSpec((1, 1), lambda i: (0, 0)),
            pl.BlockSpec((d, d), lambda i: (0, 0)),
        ],
        out_specs=pl.BlockSpec((blk, d), lambda i: (i, 0)),
        out_shape=jax.ShapeDtypeStruct((n, d), jnp.float32),
    )(dinv, p, hp, b, a, w)


def _tc_last(dinv, p, hp, b, a, blk):
    n, d = hp.shape

    def body(dv_ref, p_ref, hp_ref, b_ref, a_ref, o_ref):
        t = dv_ref[...] * (p_ref[0] + p_ref[1] + hp_ref[...]) + b_ref[...]
        o_ref[...] = jnp.where(t >= 0, t, a_ref[...] * t)

    return pl.pallas_call(
        body,
        grid=(n // blk,),
        in_specs=[
            pl.BlockSpec((blk, 1), lambda i: (i, 0)),
            pl.BlockSpec((NC, blk, d), lambda i: (0, i, 0)),
            pl.BlockSpec((blk, d), lambda i: (i, 0)),
            pl.BlockSpec((1, d), lambda i: (0, 0)),
            pl.BlockSpec((1, 1), lambda i: (0, 0)),
        ],
        out_specs=pl.BlockSpec((blk, d), lambda i: (i, 0)),
        out_shape=jax.ShapeDtypeStruct((n, d), jnp.float32),
    )(dinv, p, hp, b, a)


def kernel(x, edge_index, W1, b1, a1, W2, b2, a2):
    n, d = x.shape
    e = edge_index.shape[1]
    src = edge_index[0].astype(jnp.int32)
    dst = edge_index[1].astype(jnp.int32)

    nch = -(-e // (CH * NW * 8)) * NW * 8
    ep = nch * CH
    jw = nch // NW
    n_pad = (n // 128 + 2) * 128

    ar = jnp.arange(ep - e, dtype=jnp.int32)
    pad_src = lax.rem(ar, jnp.int32(n))
    pad_dst = n + lax.rem(ar, jnp.int32(n_pad - n))
    pk = jnp.concatenate([src | (dst << 16), pad_src | (pad_dst << 16)])
    pk_w = pk.reshape(NW, jw, CH)
    pk_cs = pk.reshape(NC, NS, jw, CH)

    blk = 1000 if n % 1000 == 0 else 8
    assert n % blk == 0

    deg_p = _sc_degree(n_pad, jw)(pk_w)
    dinv = jnp.swapaxes(_tc_dinv(deg_p, n_pad), 0, 1)[:n]

    scat = _sc_scatter(n_pad, jw, d)
    hp1 = _tc_first(dinv, x, W1, blk)
    p1 = scat(hp1, pk_cs)
    hp2 = _tc_mid(dinv, p1, hp1, b1.reshape(1, d), a1.reshape(1, 1), W2, blk)
    p2 = scat(hp2, pk_cs)
    return _tc_last(dinv, p2, hp2, b2.reshape(1, d), a2.reshape(1, 1), blk)


# speedup vs baseline: 3.0552x; 1.0023x over previous
import functools

import jax
import jax.numpy as jnp
from jax import lax
from jax.experimental import pallas as pl
from jax.experimental.pallas import tpu as pltpu
from jax.experimental.pallas import tpu_sc as plsc

NC = 2
NS = 16
L = 16
NW = NC * NS
CH = 128


def _sc_degree(n_pad, jd):
    mesh = plsc.VectorSubcoreMesh(core_axis_name="c", subcore_axis_name="s")
    gstep = 8

    @functools.partial(
        pl.kernel,
        out_type=jax.ShapeDtypeStruct((NC, NS, n_pad), jnp.float32),
        mesh=mesh,
        scratch_types=[
            pltpu.VMEM((gstep, CH), jnp.int32),
            pltpu.VMEM((n_pad,), jnp.float32),
        ],
        compiler_params=pltpu.CompilerParams(needs_layout_passes=False),
    )
    def deg_kernel(dst_hbm, out_hbm, stage_v, deg_v):
        c = lax.axis_index("c")
        s = lax.axis_index("s")
        w = c * NS + s
        zeros16 = jnp.zeros((L,), jnp.float32)

        def zb(i, carry):
            deg_v[pl.ds(i * L, L)] = zeros16
            return carry

        lax.fori_loop(0, n_pad // L, zb, 0)
        ones16 = jnp.ones((L,), jnp.float32)

        def body(r, carry):
            for k in range(CH // L):
                plsc.addupdate_scatter(
                    deg_v, [stage_v[r, pl.ds(k * L, L)]], ones16)
            return carry

        for g in range(0, jd, gstep):
            pltpu.sync_copy(dst_hbm.at[w, pl.ds(g, gstep)], stage_v)
            lax.fori_loop(0, gstep, body, 0)
        pltpu.sync_copy(deg_v, out_hbm.at[c, s])

    return deg_kernel


RD = 4


def _sc_scatter(n_pad, jw, d):
    mesh = plsc.VectorSubcoreMesh(core_axis_name="c", subcore_axis_name="s")
    rt = n_pad // NS

    @functools.partial(
        pl.kernel,
        out_type=jax.ShapeDtypeStruct((NC, n_pad, d), jnp.float32),
        mesh=mesh,
        scratch_types=[
            pltpu.VMEM((RD, CH), jnp.int32),
            pltpu.VMEM((RD, CH), jnp.int32),
            pltpu.VMEM((2, CH, d), jnp.float32),
            pltpu.VMEM_SHARED((n_pad, d), jnp.float32),
            pltpu.SemaphoreType.DMA,
            pltpu.SemaphoreType.DMA,
            pltpu.SemaphoreType.DMA,
            pltpu.SemaphoreType.DMA,
        ],
    )
    def scat_kernel(hp_hbm, src_hbm, dst_hbm, out_hbm,
                    su_v, du_v, rows_v, acc_sh, gsem, ssem, us_sem, ud_sem):
        c = lax.axis_index("c")
        s = lax.axis_index("s")

        zeros16 = jnp.zeros((L,), jnp.float32)

        def zb(i, carry):
            for k in range(d // L):
                rows_v[1, i, pl.ds(k * L, L)] = zeros16
            return carry

        lax.fori_loop(0, CH, zb, 0)

        def zb2(i, carry):
            pltpu.sync_copy(
                rows_v.at[1],
                acc_sh.at[pl.ds(pl.multiple_of(s * rt + i * CH, 8), CH)])
            return carry

        lax.fori_loop(0, rt // CH, zb2, 0)
        if rt % CH:
            pltpu.sync_copy(
                rows_v.at[1, pl.ds(0, rt % CH)],
                acc_sh.at[pl.ds(
                    pl.multiple_of(s * rt + (rt // CH) * CH, 8), rt % CH)])
        plsc.subcore_barrier()

        def stage(j):
            r = lax.rem(j, RD)
            pltpu.async_copy(src_hbm.at[c, s, j], su_v.at[r], us_sem)
            pltpu.async_copy(dst_hbm.at[c, s, j], du_v.at[r], ud_sem)

        def wait_stage(j):
            r = lax.rem(j, RD)
            pltpu.make_async_copy(src_hbm.at[c, s, j], su_v.at[r],
                                  us_sem).wait()
            pltpu.make_async_copy(dst_hbm.at[c, s, j], du_v.at[r],
                                  ud_sem).wait()

        for t in range(min(RD - 1, jw)):
            stage(t)
        wait_stage(0)
        pltpu.async_copy(hp_hbm.at[su_v.at[0]], rows_v.at[0], gsem)

        def body(j, carry):
            b = lax.rem(j, 2)
            r = lax.rem(j, RD)
            pltpu.make_async_copy(hp_hbm.at[su_v.at[r]],
                                  rows_v.at[b], gsem).wait()
            pltpu.async_copy(rows_v.at[b], acc_sh.at[du_v.at[r]], ssem,
                             add=True)

            @pl.when(j >= 1)
            def _():
                rp = lax.rem(j + RD - 1, RD)
                pltpu.make_async_copy(rows_v.at[1 - b],
                                      acc_sh.at[du_v.at[rp]], ssem).wait()

            @pl.when(j + 1 < jw)
            def _():
                wait_stage(j + 1)
                rn = lax.rem(j + 1, RD)
                pltpu.async_copy(hp_hbm.at[su_v.at[rn]],
                                 rows_v.at[1 - b], gsem)

            @pl.when(j + RD - 1 < jw)
            def _():
                stage(j + RD - 1)

            return carry

        lax.fori_loop(0, jw, body, 0)
        pltpu.make_async_copy(rows_v.at[(jw - 1) % 2],
                              acc_sh.at[du_v.at[(jw - 1) % RD]], ssem).wait()
        plsc.subcore_barrier()
        pltpu.sync_copy(
            acc_sh.at[pl.ds(pl.multiple_of(s * rt, 8), rt)],
            out_hbm.at[c, pl.ds(pl.multiple_of(s * rt, 8), rt)])

    return scat_kernel


def _tc_dinv(deg_p, n_pad):
    def body(dp_ref, o_ref):
        deg = jnp.sum(dp_ref[...], axis=(0, 1)) + 1.0
        o_ref[...] = lax.rsqrt(deg).reshape(1, n_pad)

    return pl.pallas_call(
        body,
        grid=(1,),
        in_specs=[pl.BlockSpec((NC, NS, n_pad), lambda i: (0, 0, 0))],
        out_specs=pl.BlockSpec((1, n_pad), lambda i: (0, 0)),
        out_shape=jax.ShapeDtypeStruct((1, n_pad), jnp.float32),
    )(deg_p)


def _tc_first(dinv, x, w, blk):
    n, d = x.shape

    def body(dv_ref, x_ref, w_ref, o_ref):
        h = jnp.dot(x_ref[...], w_ref[...], preferred_element_type=jnp.float32)
        o_ref[...] = h * dv_ref[...]

    return pl.pallas_call(
        body,
        grid=(n // blk,),
        in_specs=[
            pl.BlockSpec((blk, 1), lambda i: (i, 0)),
            pl.BlockSpec((blk, d), lambda i: (i, 0)),
            pl.BlockSpec((d, d), lambda i: (0, 0)),
        ],
        out_specs=pl.BlockSpec((blk, d), lambda i: (i, 0)),
        out_shape=jax.ShapeDtypeStruct((n, d), jnp.float32),
    )(dinv, x, w)


def _tc_mid(dinv, p, hp, b, a, w, blk):
    n, d = hp.shape

    def body(dv_ref, p_ref, hp_ref, b_ref, a_ref, w_ref, o_ref):
        dc = dv_ref[...]
        t = dc * (p_ref[0] + p_ref[1] + hp_ref[...]) + b_ref[...]
        u = jnp.where(t >= 0, t, a_ref[...] * t)
        h = jnp.dot(u, w_ref[...], preferred_element_type=jnp.float32)
        o_ref[...] = h * dc

    return pl.pallas_call(
        body,
        grid=(n // blk,),
        in_specs=[
            pl.BlockSpec((blk, 1), lambda i: (i, 0)),
            pl.BlockSpec((NC, blk, d), lambda i: (0, i, 0)),
            pl.BlockSpec((blk, d), lambda i: (i, 0)),
            pl.BlockSpec((1, d), lambda i: (0, 0)),
            pl.BlockSpec((1, 1), lambda i: (0, 0)),
            pl.BlockSpec((d, d), lambda i: (0, 0)),
        ],
        out_specs=pl.BlockSpec((blk, d), lambda i: (i, 0)),
        out_shape=jax.ShapeDtypeStruct((n, d), jnp.float32),
    )(dinv, p, hp, b, a, w)


def _tc_last(dinv, p, hp, b, a, blk):
    n, d = hp.shape

    def body(dv_ref, p_ref, hp_ref, b_ref, a_ref, o_ref):
        t = dv_ref[...] * (p_ref[0] + p_ref[1] + hp_ref[...]) + b_ref[...]
        o_ref[...] = jnp.where(t >= 0, t, a_ref[...] * t)

    return pl.pallas_call(
        body,
        grid=(n // blk,),
        in_specs=[
            pl.BlockSpec((blk, 1), lambda i: (i, 0)),
            pl.BlockSpec((NC, blk, d), lambda i: (0, i, 0)),
            pl.BlockSpec((blk, d), lambda i: (i, 0)),
            pl.BlockSpec((1, d), lambda i: (0, 0)),
            pl.BlockSpec((1, 1), lambda i: (0, 0)),
        ],
        out_specs=pl.BlockSpec((blk, d), lambda i: (i, 0)),
        out_shape=jax.ShapeDtypeStruct((n, d), jnp.float32),
    )(dinv, p, hp, b, a)


def kernel(x, edge_index, W1, b1, a1, W2, b2, a2):
    n, d = x.shape
    e = edge_index.shape[1]
    src = edge_index[0].astype(jnp.int32)
    dst = edge_index[1].astype(jnp.int32)

    nch = -(-e // (CH * NW * 8)) * NW * 8
    ep = nch * CH
    jw = nch // NW
    n_pad = (n // 128 + 2) * 128

    ar = jnp.arange(ep - e, dtype=jnp.int32)
    pad_src = lax.rem(ar, jnp.int32(n))
    pad_dst = n + lax.rem(ar, jnp.int32(n_pad - n))
    srcp = jnp.concatenate([src, pad_src])
    dstp = jnp.concatenate([dst, pad_dst])
    dst_w = dstp.reshape(NW, jw, CH)
    src_cs = srcp.reshape(NC, NS, jw, CH)
    dst_cs = dstp.reshape(NC, NS, jw, CH)

    blk = 1000 if n % 1000 == 0 else 8
    assert n % blk == 0

    deg_p = _sc_degree(n_pad, jw)(dst_w)
    dinv = jnp.swapaxes(_tc_dinv(deg_p, n_pad), 0, 1)[:n]

    scat = _sc_scatter(n_pad, jw, d)
    hp1 = _tc_first(dinv, x, W1, blk)
    p1 = scat(hp1, src_cs, dst_cs)
    hp2 = _tc_mid(dinv, p1, hp1, b1.reshape(1, d), a1.reshape(1, 1), W2, blk)
    p2 = scat(hp2, src_cs, dst_cs)
    return _tc_last(dinv, p2, hp2, b2.reshape(1, d), a2.reshape(1, 1), blk)


# speedup vs baseline: 3.0625x; 1.0024x over previous
import functools

import jax
import jax.numpy as jnp
import numpy as np
from jax import lax
from jax.experimental import pallas as pl
from jax.experimental.pallas import tpu as pltpu
from jax.experimental.pallas import tpu_sc as plsc

NC = 2
NS = 16
L = 16
NW = NC * NS
CH = 128


def _sc_degree(n_pad, jd):
    mesh = plsc.VectorSubcoreMesh(core_axis_name="c", subcore_axis_name="s")
    gstep = 8

    @functools.partial(
        pl.kernel,
        out_type=jax.ShapeDtypeStruct((NC, NS, n_pad), jnp.float32),
        mesh=mesh,
        scratch_types=[
            pltpu.VMEM((gstep, CH), jnp.int32),
            pltpu.VMEM((n_pad,), jnp.float32),
        ],
        compiler_params=pltpu.CompilerParams(needs_layout_passes=False),
    )
    def deg_kernel(dst_hbm, out_hbm, stage_v, deg_v):
        c = lax.axis_index("c")
        s = lax.axis_index("s")
        w = c * NS + s
        zeros16 = jnp.zeros((L,), jnp.float32)

        def zb(i, carry):
            deg_v[pl.ds(i * L, L)] = zeros16
            return carry

        lax.fori_loop(0, n_pad // L, zb, 0)
        ones16 = jnp.ones((L,), jnp.float32)

        def body(r, carry):
            for k in range(CH // L):
                plsc.addupdate_scatter(
                    deg_v, [stage_v[r, pl.ds(k * L, L)]], ones16)
            return carry

        for g in range(0, jd, gstep):
            pltpu.sync_copy(dst_hbm.at[w, pl.ds(g, gstep)], stage_v)
            lax.fori_loop(0, gstep, body, 0)
        pltpu.sync_copy(deg_v, out_hbm.at[c, s])

    return deg_kernel


RD = 4


def _sc_scatter(n_pad, jw, d):
    mesh = plsc.VectorSubcoreMesh(core_axis_name="c", subcore_axis_name="s")
    rt = n_pad // NS

    @functools.partial(
        pl.kernel,
        out_type=jax.ShapeDtypeStruct((NC, n_pad, d), jnp.float32),
        mesh=mesh,
        scratch_types=[
            pltpu.VMEM((RD, CH), jnp.int32),
            pltpu.VMEM((RD, CH), jnp.int32),
            pltpu.VMEM((2, CH, d), jnp.float32),
            pltpu.VMEM_SHARED((n_pad, d), jnp.float32),
            pltpu.SemaphoreType.DMA,
            pltpu.SemaphoreType.DMA,
            pltpu.SemaphoreType.DMA,
            pltpu.SemaphoreType.DMA,
        ],
    )
    def scat_kernel(hp_hbm, src_hbm, dst_hbm, out_hbm,
                    su_v, du_v, rows_v, acc_sh, gsem, ssem, us_sem, ud_sem):
        c = lax.axis_index("c")
        s = lax.axis_index("s")

        zeros16 = jnp.zeros((L,), jnp.float32)

        def zb(i, carry):
            for k in range(d // L):
                rows_v[1, i, pl.ds(k * L, L)] = zeros16
            return carry

        lax.fori_loop(0, CH, zb, 0)

        def zb2(i, carry):
            pltpu.sync_copy(
                rows_v.at[1],
                acc_sh.at[pl.ds(pl.multiple_of(s * rt + i * CH, 8), CH)])
            return carry

        lax.fori_loop(0, rt // CH, zb2, 0)
        if rt % CH:
            pltpu.sync_copy(
                rows_v.at[1, pl.ds(0, rt % CH)],
                acc_sh.at[pl.ds(
                    pl.multiple_of(s * rt + (rt // CH) * CH, 8), rt % CH)])
        plsc.subcore_barrier()

        def stage(j):
            r = lax.rem(j, RD)
            pltpu.async_copy(src_hbm.at[c, s, j], su_v.at[r], us_sem)
            pltpu.async_copy(dst_hbm.at[c, s, j], du_v.at[r], ud_sem)

        def wait_stage(j):
            r = lax.rem(j, RD)
            pltpu.make_async_copy(src_hbm.at[c, s, j], su_v.at[r],
                                  us_sem).wait()
            pltpu.make_async_copy(dst_hbm.at[c, s, j], du_v.at[r],
                                  ud_sem).wait()

        for t in range(min(RD - 1, jw)):
            stage(t)
        wait_stage(0)
        pltpu.async_copy(hp_hbm.at[su_v.at[0]], rows_v.at[0], gsem)

        def body(j, carry):
            b = lax.rem(j, 2)
            r = lax.rem(j, RD)
            pltpu.make_async_copy(hp_hbm.at[su_v.at[r]],
                                  rows_v.at[b], gsem).wait()
            pltpu.async_copy(rows_v.at[b], acc_sh.at[du_v.at[r]], ssem,
                             add=True)

            @pl.when(j >= 1)
            def _():
                rp = lax.rem(j + RD - 1, RD)
                pltpu.make_async_copy(rows_v.at[1 - b],
                                      acc_sh.at[du_v.at[rp]], ssem).wait()

            @pl.when(j + 1 < jw)
            def _():
                wait_stage(j + 1)
                rn = lax.rem(j + 1, RD)
                pltpu.async_copy(hp_hbm.at[su_v.at[rn]],
                                 rows_v.at[1 - b], gsem)

            @pl.when(j + RD - 1 < jw)
            def _():
                stage(j + RD - 1)

            return carry

        lax.fori_loop(0, jw, body, 0)
        pltpu.make_async_copy(rows_v.at[(jw - 1) % 2],
                              acc_sh.at[du_v.at[(jw - 1) % RD]], ssem).wait()
        plsc.subcore_barrier()
        pltpu.sync_copy(
            acc_sh.at[pl.ds(pl.multiple_of(s * rt, 8), rt)],
            out_hbm.at[c, pl.ds(pl.multiple_of(s * rt, 8), rt)])

    return scat_kernel


def _tc_dinv(deg_p, n_pad):
    def body(dp_ref, o_ref):
        deg = jnp.sum(dp_ref[...], axis=(0, 1)) + 1.0
        o_ref[...] = lax.rsqrt(deg).reshape(1, n_pad)

    return pl.pallas_call(
        body,
        grid=(1,),
        in_specs=[pl.BlockSpec((NC, NS, n_pad), lambda i: (0, 0, 0))],
        out_specs=pl.BlockSpec((1, n_pad), lambda i: (0, 0)),
        out_shape=jax.ShapeDtypeStruct((1, n_pad), jnp.float32),
    )(deg_p)


def _tc_matmul(x, w, blk):
    n, d = x.shape

    def body(x_ref, w_ref, o_ref):
        o_ref[...] = jnp.dot(x_ref[...], w_ref[...],
                             preferred_element_type=jnp.float32)

    return pl.pallas_call(
        body,
        grid=(n // blk,),
        in_specs=[
            pl.BlockSpec((blk, d), lambda i: (i, 0)),
            pl.BlockSpec((d, d), lambda i: (0, 0)),
        ],
        out_specs=pl.BlockSpec((blk, d), lambda i: (i, 0)),
        out_shape=jax.ShapeDtypeStruct((n, d), jnp.float32),
    )(x, w)


def _tc_scale(dinv, h, blk):
    n, d = h.shape

    def body(dv_ref, h_ref, o_ref):
        o_ref[...] = h_ref[...] * dv_ref[...]

    return pl.pallas_call(
        body,
        grid=(n // blk,),
        in_specs=[
            pl.BlockSpec((blk, 1), lambda i: (i, 0)),
            pl.BlockSpec((blk, d), lambda i: (i, 0)),
        ],
        out_specs=pl.BlockSpec((blk, d), lambda i: (i, 0)),
        out_shape=jax.ShapeDtypeStruct((n, d), jnp.float32),
    )(dinv, h)


def _tc_mid(dinv, p, hp, b, a, w, blk):
    n, d = hp.shape

    def body(dv_ref, p_ref, hp_ref, b_ref, a_ref, w_ref, o_ref):
        dc = dv_ref[...]
        t = dc * (p_ref[0] + p_ref[1] + hp_ref[...]) + b_ref[...]
        u = jnp.where(t >= 0, t, a_ref[...] * t)
        h = jnp.dot(u, w_ref[...], preferred_element_type=jnp.float32)
        o_ref[...] = h * dc

    return pl.pallas_call(
        body,
        grid=(n // blk,),
        in_specs=[
            pl.BlockSpec((blk, 1), lambda i: (i, 0)),
            pl.BlockSpec((NC, blk, d), lambda i: (0, i, 0)),
            pl.BlockSpec((blk, d), lambda i: (i, 0)),
            pl.BlockSpec((1, d), lambda i: (0, 0)),
            pl.BlockSpec((1, 1), lambda i: (0, 0)),
            pl.BlockSpec((d, d), lambda i: (0, 0)),
        ],
        out_specs=pl.BlockSpec((blk, d), lambda i: (i, 0)),
        out_shape=jax.ShapeDtypeStruct((n, d), jnp.float32),
    )(dinv, p, hp, b, a, w)


def _tc_last(dinv, p, hp, b, a, blk):
    n, d = hp.shape

    def body(dv_ref, p_ref, hp_ref, b_ref, a_ref, o_ref):
        t = dv_ref[...] * (p_ref[0] + p_ref[1] + hp_ref[...]) + b_ref[...]
        o_ref[...] = jnp.where(t >= 0, t, a_ref[...] * t)

    return pl.pallas_call(
        body,
        grid=(n // blk,),
        in_specs=[
            pl.BlockSpec((blk, 1), lambda i: (i, 0)),
            pl.BlockSpec((NC, blk, d), lambda i: (0, i, 0)),
            pl.BlockSpec((blk, d), lambda i: (i, 0)),
            pl.BlockSpec((1, d), lambda i: (0, 0)),
            pl.BlockSpec((1, 1), lambda i: (0, 0)),
        ],
        out_specs=pl.BlockSpec((blk, d), lambda i: (i, 0)),
        out_shape=jax.ShapeDtypeStruct((n, d), jnp.float32),
    )(dinv, p, hp, b, a)


def kernel(x, edge_index, W1, b1, a1, W2, b2, a2):
    n, d = x.shape
    e = edge_index.shape[1]
    src = edge_index[0].astype(jnp.int32)
    dst = edge_index[1].astype(jnp.int32)

    nch = -(-e // (CH * NW * 8)) * NW * 8
    ep = nch * CH
    jw = nch // NW
    n_pad = (n // 128 + 2) * 128

    ar = np.arange(ep - e, dtype=np.int32)
    pad_src = jnp.asarray(ar % np.int32(n))
    pad_dst = jnp.asarray(n + ar % np.int32(n_pad - n))
    srcp = jnp.concatenate([src, pad_src])
    dstp = jnp.concatenate([dst, pad_dst])
    dst_w = dstp.reshape(NW, jw, CH)
    src_cs = srcp.reshape(NC, NS, jw, CH)
    dst_cs = dstp.reshape(NC, NS, jw, CH)

    blk = 1000 if n % 1000 == 0 else 8
    assert n % blk == 0

    deg_p = _sc_degree(n_pad, jw)(dst_w)
    h1 = _tc_matmul(x, W1, blk)
    dinv = jnp.swapaxes(_tc_dinv(deg_p, n_pad), 0, 1)[:n]

    scat = _sc_scatter(n_pad, jw, d)
    hp1 = _tc_scale(dinv, h1, blk)
    p1 = scat(hp1, src_cs, dst_cs)
    hp2 = _tc_mid(dinv, p1, hp1, b1.reshape(1, d), a1.reshape(1, 1), W2, blk)
    p2 = scat(hp2, src_cs, dst_cs)
    return _tc_last(dinv, p2, hp2, b2.reshape(1, d), a2.reshape(1, 1), blk)
